# baseline (speedup 1.0000x reference)
import functools

import jax
import jax.numpy as jnp
from jax import lax
from jax.experimental import pallas as pl
from jax.experimental.pallas import tpu as pltpu
from jax.experimental.pallas import tpu_sc as plsc

BATCH = 16384
EMBED = 64
H1 = 128
H2 = 64


@functools.cache
def _build_gather():
    info = plsc.get_sparse_core_info()
    nc, ns = info.num_cores, info.num_subcores
    nw = nc * ns
    bpw = BATCH // nw

    mesh = plsc.VectorSubcoreMesh(core_axis_name="c", subcore_axis_name="s")

    @functools.partial(
        pl.kernel,
        mesh=mesh,
        compiler_params=pltpu.CompilerParams(use_tc_tiling_on_sc=False),
        out_type=[
            jax.ShapeDtypeStruct((BATCH, EMBED), jnp.float32),
            jax.ShapeDtypeStruct((BATCH, EMBED), jnp.float32),
        ],
        scratch_types=[
            pltpu.VMEM((bpw,), jnp.int32),
            pltpu.VMEM((bpw,), jnp.int32),
            pltpu.VMEM((bpw, EMBED), jnp.float32),
            pltpu.VMEM((bpw, EMBED), jnp.float32),
            pltpu.SemaphoreType.DMA,
            pltpu.SemaphoreType.DMA,
        ],
    )
    def gather(uid_hbm, iid_hbm, ut_hbm, it_hbm, u_out, i_out,
               uidx_v, iidx_v, urows_v, irows_v, usem, isem):
        wid = lax.axis_index("s") * nc + lax.axis_index("c")
        base = wid * bpw
        pltpu.sync_copy(uid_hbm.at[pl.ds(base, bpw)], uidx_v)
        pltpu.sync_copy(iid_hbm.at[pl.ds(base, bpw)], iidx_v)
        ucp = pltpu.async_copy(ut_hbm.at[uidx_v], urows_v, usem)
        icp = pltpu.async_copy(it_hbm.at[iidx_v], irows_v, isem)
        ucp.wait()
        pltpu.sync_copy(urows_v, u_out.at[pl.ds(base, bpw)])
        icp.wait()
        pltpu.sync_copy(irows_v, i_out.at[pl.ds(base, bpw)])

    return gather


def _mlp_body(u_ref, i_ref, w1_ref, b1_ref, w2_ref, b2_ref, w3_ref, b3_ref,
              o_ref):
    h = jnp.dot(u_ref[...], w1_ref[:EMBED, :],
                preferred_element_type=jnp.float32)
    h = h + jnp.dot(i_ref[...], w1_ref[EMBED:, :],
                    preferred_element_type=jnp.float32)
    h = jnp.maximum(h + b1_ref[...], 0.0)
    h = jnp.maximum(
        jnp.dot(h, w2_ref[...], preferred_element_type=jnp.float32)
        + b2_ref[...], 0.0)
    z = jnp.sum(h * w3_ref[...], axis=1) + b3_ref[0, 0]
    o_ref[...] = 1.0 / (1.0 + jnp.exp(-z))


def _mlp(u_emb, i_emb, W1, b1r, W2, b2r, w3r, b3r, block=2048):
    grid = BATCH // block
    return pl.pallas_call(
        _mlp_body,
        grid=(grid,),
        in_specs=[
            pl.BlockSpec((block, EMBED), lambda b: (b, 0)),
            pl.BlockSpec((block, EMBED), lambda b: (b, 0)),
            pl.BlockSpec((2 * EMBED, H1), lambda b: (0, 0)),
            pl.BlockSpec((1, H1), lambda b: (0, 0)),
            pl.BlockSpec((H1, H2), lambda b: (0, 0)),
            pl.BlockSpec((1, H2), lambda b: (0, 0)),
            pl.BlockSpec((1, H2), lambda b: (0, 0)),
            pl.BlockSpec((1, 1), lambda b: (0, 0)),
        ],
        out_specs=pl.BlockSpec((block,), lambda b: (b,)),
        out_shape=jax.ShapeDtypeStruct((BATCH,), jnp.float32),
    )(u_emb, i_emb, W1, b1r, W2, b2r, w3r, b3r)


def kernel(user_ids, item_ids, user_table, item_table, W1, b1, W2, b2, W3, b3):
    uid = user_ids.astype(jnp.int32)
    iid = item_ids.astype(jnp.int32)
    u_emb, i_emb = _build_gather()(uid, iid, user_table, item_table)
    return _mlp(u_emb, i_emb, W1, b1.reshape(1, H1), W2, b2.reshape(1, H2),
                W3.reshape(1, H2), b3.reshape(1, 1))


# speedup vs baseline: 1.5715x; 1.5715x over previous
import functools

import jax
import jax.numpy as jnp
from jax import lax
from jax.experimental import pallas as pl
from jax.experimental.pallas import tpu as pltpu
from jax.experimental.pallas import tpu_sc as plsc

BATCH = 16384
EMBED = 64
H1 = 128
H2 = 64


@functools.cache
def _build_gather():
    info = plsc.get_sparse_core_info()
    nc, ns = info.num_cores, info.num_subcores
    nw = nc * ns
    bpw = BATCH // nw

    mesh = plsc.VectorSubcoreMesh(core_axis_name="c", subcore_axis_name="s")

    @functools.partial(
        pl.kernel,
        mesh=mesh,
        out_type=[
            jax.ShapeDtypeStruct((BATCH, EMBED), jnp.float32),
            jax.ShapeDtypeStruct((BATCH, EMBED), jnp.float32),
        ],
        scratch_types=[
            pltpu.VMEM((bpw,), jnp.int32),
            pltpu.VMEM((bpw,), jnp.int32),
            pltpu.VMEM((bpw, EMBED), jnp.float32),
            pltpu.SemaphoreType.DMA,
        ],
    )
    def gather(uid_hbm, iid_hbm, ut_hbm, it_hbm, u_out, i_out,
               uidx_v, iidx_v, rows_v, sem):
        wid = lax.axis_index("s") * nc + lax.axis_index("c")
        base = wid * bpw
        pltpu.sync_copy(uid_hbm.at[pl.ds(base, bpw)], uidx_v)
        pltpu.sync_copy(iid_hbm.at[pl.ds(base, bpw)], iidx_v)

        for idx_v, tab, out in ((uidx_v, ut_hbm, u_out),
                                (iidx_v, it_hbm, i_out)):
            def fire(j, _, idx_v=idx_v, tab=tab):
                vec = idx_v[pl.ds(j * 16, 16)]
                for k in range(16):
                    pltpu.make_async_copy(
                        tab.at[pl.ds(vec[k], 1), :],
                        rows_v.at[pl.ds(j * 16 + k, 1), :], sem).start()
                return 0

            lax.fori_loop(0, bpw // 16, fire, 0)

            def drain(i, _, tab=tab):
                pltpu.make_async_copy(
                    tab.at[pl.ds(0, 1), :], rows_v.at[pl.ds(0, 1), :],
                    sem).wait()
                return 0

            lax.fori_loop(0, bpw, drain, 0)
            pltpu.sync_copy(rows_v, out.at[pl.ds(base, bpw)])

    return gather


def _mlp_body(u_ref, i_ref, w1_ref, b1_ref, w2_ref, b2_ref, w3_ref, b3_ref,
              o_ref):
    h = jnp.dot(u_ref[...], w1_ref[:EMBED, :],
                preferred_element_type=jnp.float32)
    h = h + jnp.dot(i_ref[...], w1_ref[EMBED:, :],
                    preferred_element_type=jnp.float32)
    h = jnp.maximum(h + b1_ref[...], 0.0)
    h = jnp.maximum(
        jnp.dot(h, w2_ref[...], preferred_element_type=jnp.float32)
        + b2_ref[...], 0.0)
    z = jnp.sum(h * w3_ref[...], axis=1) + b3_ref[0, 0]
    o_ref[...] = 1.0 / (1.0 + jnp.exp(-z))


def _mlp(u_emb, i_emb, W1, b1r, W2, b2r, w3r, b3r, block=2048):
    grid = BATCH // block
    return pl.pallas_call(
        _mlp_body,
        grid=(grid,),
        in_specs=[
            pl.BlockSpec((block, EMBED), lambda b: (b, 0)),
            pl.BlockSpec((block, EMBED), lambda b: (b, 0)),
            pl.BlockSpec((2 * EMBED, H1), lambda b: (0, 0)),
            pl.BlockSpec((1, H1), lambda b: (0, 0)),
            pl.BlockSpec((H1, H2), lambda b: (0, 0)),
            pl.BlockSpec((1, H2), lambda b: (0, 0)),
            pl.BlockSpec((1, H2), lambda b: (0, 0)),
            pl.BlockSpec((1, 1), lambda b: (0, 0)),
        ],
        out_specs=pl.BlockSpec((block,), lambda b: (b,)),
        out_shape=jax.ShapeDtypeStruct((BATCH,), jnp.float32),
    )(u_emb, i_emb, W1, b1r, W2, b2r, w3r, b3r)


def kernel(user_ids, item_ids, user_table, item_table, W1, b1, W2, b2, W3, b3):
    uid = user_ids.astype(jnp.int32)
    iid = item_ids.astype(jnp.int32)
    u_emb, i_emb = _build_gather()(uid, iid, user_table, item_table)
    return _mlp(u_emb, i_emb, W1, b1.reshape(1, H1), W2, b2.reshape(1, H2),
                W3.reshape(1, H2), b3.reshape(1, 1))


# speedup vs baseline: 2.3284x; 1.4816x over previous
import functools

import jax
import jax.numpy as jnp
from jax import lax
from jax.experimental import pallas as pl
from jax.experimental.pallas import tpu as pltpu
from jax.experimental.pallas import tpu_sc as plsc

BATCH = 16384
EMBED = 64
H1 = 128
H2 = 64
RING = 4


@functools.cache
def _build_gather():
    info = plsc.get_sparse_core_info()
    nc, ns = info.num_cores, info.num_subcores
    nw = nc * ns
    bpw = BATCH // nw
    nrounds = bpw // 16

    mesh = plsc.VectorSubcoreMesh(core_axis_name="c", subcore_axis_name="s")

    @functools.partial(
        pl.kernel,
        mesh=mesh,
        compiler_params=pltpu.CompilerParams(needs_layout_passes=False),
        out_type=[
            jax.ShapeDtypeStruct((BATCH, EMBED), jnp.float32),
            jax.ShapeDtypeStruct((BATCH, EMBED), jnp.float32),
        ],
        scratch_types=[
            pltpu.VMEM((bpw,), jnp.int32),
            pltpu.VMEM((bpw,), jnp.int32),
            pltpu.VMEM((bpw, EMBED), jnp.float32),
        ] + [pltpu.VMEM((EMBED, 128), jnp.float32) for _ in range(RING)]
          + [pltpu.SemaphoreType.DMA for _ in range(RING)],
    )
    def gather(uid_hbm, iid_hbm, utt_hbm, itt_hbm, u_out, i_out,
               uidx_v, iidx_v, rows_v, *bufsem):
        bufs = bufsem[:RING]
        sems = bufsem[RING:]
        wid = lax.axis_index("s") * nc + lax.axis_index("c")
        base = wid * bpw
        pltpu.sync_copy(uid_hbm.at[pl.ds(base, bpw)], uidx_v)
        pltpu.sync_copy(iid_hbm.at[pl.ds(base, bpw)], iidx_v)
        iota16 = lax.iota(jnp.int32, 16)

        def fire(tab, ident, slot):
            col = pl.multiple_of((ident >> 7) << 7, 128)
            pltpu.make_async_copy(
                tab.at[:, pl.ds(col, 128)], bufs[slot], sems[slot]).start()

        def wait(tab, slot):
            pltpu.make_async_copy(
                tab.at[:, pl.ds(0, 128)], bufs[slot], sems[slot]).wait()

        def extract(ident, g, slot):
            lane = jnp.full((16,), ident & 127, jnp.int32)
            gv = jnp.full((16,), g, jnp.int32)
            for b in range(EMBED // 16):
                vals = plsc.load_gather(bufs[slot], [iota16 + b * 16, lane])
                plsc.store_scatter(rows_v, [gv, iota16 + b * 16], vals)

        for idx_v, tab, out in ((uidx_v, utt_hbm, u_out),
                                (iidx_v, itt_hbm, i_out)):
            vec0 = idx_v[pl.ds(0, 16)]
            for k in range(RING):
                fire(tab, vec0[k], k)

            def round_(j, _, idx_v=idx_v, tab=tab):
                vec = idx_v[pl.ds(j * 16, 16)]
                for k in range(16):
                    slot = k % RING
                    wait(tab, slot)
                    extract(vec[k], j * 16 + k, slot)
                    if k < 16 - RING:
                        fire(tab, vec[k + RING], slot)
                    else:
                        @pl.when(j < nrounds - 1)
                        def _(k=k, slot=slot, tab=tab, idx_v=idx_v, j=j):
                            vecn = idx_v[pl.ds((j + 1) * 16, 16)]
                            fire(tab, vecn[k - (16 - RING)], slot)
                return 0

            lax.fori_loop(0, nrounds, round_, 0)
            pltpu.sync_copy(rows_v, out.at[pl.ds(base, bpw)])

    return gather


def _mlp_body(u_ref, i_ref, w1_ref, b1_ref, w2_ref, b2_ref, w3_ref, b3_ref,
              o_ref):
    h = jnp.dot(u_ref[...], w1_ref[:EMBED, :],
                preferred_element_type=jnp.float32)
    h = h + jnp.dot(i_ref[...], w1_ref[EMBED:, :],
                    preferred_element_type=jnp.float32)
    h = jnp.maximum(h + b1_ref[...], 0.0)
    h = jnp.maximum(
        jnp.dot(h, w2_ref[...], preferred_element_type=jnp.float32)
        + b2_ref[...], 0.0)
    z = jnp.sum(h * w3_ref[...], axis=1) + b3_ref[0, 0]
    o_ref[...] = 1.0 / (1.0 + jnp.exp(-z))


def _mlp(u_emb, i_emb, W1, b1r, W2, b2r, w3r, b3r, block=2048):
    grid = BATCH // block
    return pl.pallas_call(
        _mlp_body,
        grid=(grid,),
        in_specs=[
            pl.BlockSpec((block, EMBED), lambda b: (b, 0)),
            pl.BlockSpec((block, EMBED), lambda b: (b, 0)),
            pl.BlockSpec((2 * EMBED, H1), lambda b: (0, 0)),
            pl.BlockSpec((1, H1), lambda b: (0, 0)),
            pl.BlockSpec((H1, H2), lambda b: (0, 0)),
            pl.BlockSpec((1, H2), lambda b: (0, 0)),
            pl.BlockSpec((1, H2), lambda b: (0, 0)),
            pl.BlockSpec((1, 1), lambda b: (0, 0)),
        ],
        out_specs=pl.BlockSpec((block,), lambda b: (b,)),
        out_shape=jax.ShapeDtypeStruct((BATCH,), jnp.float32),
    )(u_emb, i_emb, W1, b1r, W2, b2r, w3r, b3r)


def kernel(user_ids, item_ids, user_table, item_table, W1, b1, W2, b2, W3, b3):
    uid = user_ids.astype(jnp.int32)
    iid = item_ids.astype(jnp.int32)
    u_emb, i_emb = _build_gather()(uid, iid, user_table.T, item_table.T)
    return _mlp(u_emb, i_emb, W1, b1.reshape(1, H1), W2, b2.reshape(1, H2),
                W3.reshape(1, H2), b3.reshape(1, 1))


# speedup vs baseline: 2.7287x; 1.1719x over previous
import functools

import jax
import jax.numpy as jnp
from jax import lax
from jax.experimental import pallas as pl
from jax.experimental.pallas import tpu as pltpu
from jax.experimental.pallas import tpu_sc as plsc

BATCH = 16384
EMBED = 64
H1 = 128
H2 = 64
RING = 8
CHUNK = 256


@functools.cache
def _build_gather():
    info = plsc.get_sparse_core_info()
    nc, ns = info.num_cores, info.num_subcores
    nw = nc * ns
    bpw = BATCH // nw
    nchunks = bpw // CHUNK
    nrounds = CHUNK // 16

    mesh = plsc.VectorSubcoreMesh(core_axis_name="c", subcore_axis_name="s")

    @functools.partial(
        pl.kernel,
        mesh=mesh,
        compiler_params=pltpu.CompilerParams(needs_layout_passes=False),
        out_type=[
            jax.ShapeDtypeStruct((BATCH, EMBED), jnp.float32),
            jax.ShapeDtypeStruct((BATCH, EMBED), jnp.float32),
        ],
        scratch_types=[
            pltpu.VMEM((bpw,), jnp.int32),
            pltpu.VMEM((bpw,), jnp.int32),
            pltpu.VMEM((CHUNK, EMBED), jnp.float32),
        ] + [pltpu.VMEM((EMBED, 128), jnp.float32) for _ in range(RING)]
          + [pltpu.SemaphoreType.DMA for _ in range(RING)],
    )
    def gather(uid_hbm, iid_hbm, utt_hbm, itt_hbm, u_out, i_out,
               uidx_v, iidx_v, rows_v, *bufsem):
        bufs = bufsem[:RING]
        sems = bufsem[RING:]
        wid = lax.axis_index("s") * nc + lax.axis_index("c")
        base = wid * bpw
        pltpu.sync_copy(uid_hbm.at[pl.ds(base, bpw)], uidx_v)
        pltpu.sync_copy(iid_hbm.at[pl.ds(base, bpw)], iidx_v)
        iota16 = lax.iota(jnp.int32, 16)

        def fire(tab, ident, slot):
            col = pl.multiple_of((ident >> 7) << 7, 128)
            pltpu.make_async_copy(
                tab.at[:, pl.ds(col, 128)], bufs[slot], sems[slot]).start()

        def wait(tab, slot):
            pltpu.make_async_copy(
                tab.at[:, pl.ds(0, 128)], bufs[slot], sems[slot]).wait()

        def extract(ident, g, slot):
            lane = jnp.full((16,), ident & 127, jnp.int32)
            gv = jnp.full((16,), g, jnp.int32)
            for b in range(EMBED // 16):
                vals = plsc.load_gather(bufs[slot], [iota16 + b * 16, lane])
                plsc.store_scatter(rows_v, [gv, iota16 + b * 16], vals)

        for idx_v, tab, out in ((uidx_v, utt_hbm, u_out),
                                (iidx_v, itt_hbm, i_out)):
            for cc in range(nchunks):
                off = cc * CHUNK
                vec0 = idx_v[pl.ds(off, 16)]
                for k in range(RING):
                    fire(tab, vec0[k], k)

                def round_(j, _, idx_v=idx_v, tab=tab, off=off):
                    vec = idx_v[pl.ds(off + j * 16, 16)]
                    for k in range(16):
                        slot = k % RING
                        wait(tab, slot)
                        extract(vec[k], j * 16 + k, slot)
                        if k < 16 - RING:
                            fire(tab, vec[k + RING], slot)
                        else:
                            @pl.when(j < nrounds - 1)
                            def _(k=k, slot=slot, tab=tab, idx_v=idx_v,
                                  j=j, off=off):
                                vecn = idx_v[pl.ds(off + (j + 1) * 16, 16)]
                                fire(tab, vecn[k - (16 - RING)], slot)
                    return 0

                lax.fori_loop(0, nrounds, round_, 0)
                pltpu.sync_copy(rows_v, out.at[pl.ds(base + off, CHUNK)])

    return gather


def _mlp_body(u_ref, i_ref, w1_ref, b1_ref, w2_ref, b2_ref, w3_ref, b3_ref,
              o_ref):
    h = jnp.dot(u_ref[...], w1_ref[:EMBED, :],
                preferred_element_type=jnp.float32)
    h = h + jnp.dot(i_ref[...], w1_ref[EMBED:, :],
                    preferred_element_type=jnp.float32)
    h = jnp.maximum(h + b1_ref[...], 0.0)
    h = jnp.maximum(
        jnp.dot(h, w2_ref[...], preferred_element_type=jnp.float32)
        + b2_ref[...], 0.0)
    z = jnp.sum(h * w3_ref[...], axis=1) + b3_ref[0, 0]
    o_ref[...] = 1.0 / (1.0 + jnp.exp(-z))


def _mlp(u_emb, i_emb, W1, b1r, W2, b2r, w3r, b3r, block=2048):
    grid = BATCH // block
    return pl.pallas_call(
        _mlp_body,
        grid=(grid,),
        in_specs=[
            pl.BlockSpec((block, EMBED), lambda b: (b, 0)),
            pl.BlockSpec((block, EMBED), lambda b: (b, 0)),
            pl.BlockSpec((2 * EMBED, H1), lambda b: (0, 0)),
            pl.BlockSpec((1, H1), lambda b: (0, 0)),
            pl.BlockSpec((H1, H2), lambda b: (0, 0)),
            pl.BlockSpec((1, H2), lambda b: (0, 0)),
            pl.BlockSpec((1, H2), lambda b: (0, 0)),
            pl.BlockSpec((1, 1), lambda b: (0, 0)),
        ],
        out_specs=pl.BlockSpec((block,), lambda b: (b,)),
        out_shape=jax.ShapeDtypeStruct((BATCH,), jnp.float32),
    )(u_emb, i_emb, W1, b1r, W2, b2r, w3r, b3r)


def kernel(user_ids, item_ids, user_table, item_table, W1, b1, W2, b2, W3, b3):
    uid = user_ids.astype(jnp.int32)
    iid = item_ids.astype(jnp.int32)
    u_emb, i_emb = _build_gather()(uid, iid, user_table.T, item_table.T)
    return _mlp(u_emb, i_emb, W1, b1.reshape(1, H1), W2, b2.reshape(1, H2),
                W3.reshape(1, H2), b3.reshape(1, 1))


# speedup vs baseline: 2.7482x; 1.0072x over previous
import functools

import jax
import jax.numpy as jnp
from jax import lax
from jax.experimental import pallas as pl
from jax.experimental.pallas import tpu as pltpu
from jax.experimental.pallas import tpu_sc as plsc

BATCH = 16384
EMBED = 64
H1 = 128
H2 = 64
RING = 8
CHUNK = 256


@functools.cache
def _build_gather():
    info = plsc.get_sparse_core_info()
    nc, ns = info.num_cores, info.num_subcores
    nw = nc * ns
    bpw = BATCH // nw
    nchunks = bpw // CHUNK
    nrounds = CHUNK // 16

    mesh = plsc.VectorSubcoreMesh(core_axis_name="c", subcore_axis_name="s")

    @functools.partial(
        pl.kernel,
        mesh=mesh,
        compiler_params=pltpu.CompilerParams(needs_layout_passes=False),
        out_type=[
            jax.ShapeDtypeStruct((BATCH, EMBED), jnp.float32),
            jax.ShapeDtypeStruct((BATCH, EMBED), jnp.float32),
        ],
        scratch_types=[
            pltpu.VMEM((bpw,), jnp.int32),
            pltpu.VMEM((bpw,), jnp.int32),
            pltpu.VMEM((CHUNK, EMBED), jnp.float32),
        ] + [pltpu.VMEM((EMBED, 128), jnp.float32) for _ in range(RING)]
          + [pltpu.SemaphoreType.DMA for _ in range(RING)],
    )
    def gather(uid_hbm, iid_hbm, utt_hbm, itt_hbm, u_out, i_out,
               uidx_v, iidx_v, rows_v, *bufsem):
        bufs = bufsem[:RING]
        sems = bufsem[RING:]
        wid = lax.axis_index("s") * nc + lax.axis_index("c")
        base = wid * bpw
        pltpu.sync_copy(uid_hbm.at[pl.ds(base, bpw)], uidx_v)
        pltpu.sync_copy(iid_hbm.at[pl.ds(base, bpw)], iidx_v)
        iota16 = lax.iota(jnp.int32, 16)

        def fire(tab, ident, slot):
            col = pl.multiple_of((ident >> 7) << 7, 128)
            pltpu.make_async_copy(
                tab.at[:, pl.ds(col, 128)], bufs[slot], sems[slot]).start()

        def wait(tab, slot):
            pltpu.make_async_copy(
                tab.at[:, pl.ds(0, 128)], bufs[slot], sems[slot]).wait()

        def extract(ident, g, slot):
            lane = jnp.full((16,), ident & 127, jnp.int32)
            gv = jnp.full((16,), g, jnp.int32)
            for b in range(EMBED // 16):
                vals = plsc.load_gather(bufs[slot], [iota16 + b * 16, lane])
                plsc.store_scatter(rows_v, [gv, iota16 + b * 16], vals)

        segs = [(uidx_v, utt_hbm, u_out, cc * CHUNK)
                for cc in range(nchunks)]
        segs += [(iidx_v, itt_hbm, i_out, cc * CHUNK)
                 for cc in range(nchunks)]

        vec0 = uidx_v[pl.ds(0, 16)]
        for k in range(RING):
            fire(utt_hbm, vec0[k], k)

        for si, (idx_v, tab, out, off) in enumerate(segs):
            nxt = segs[si + 1] if si + 1 < len(segs) else None

            def round_(j, _, idx_v=idx_v, tab=tab, off=off, nxt=nxt):
                vec = idx_v[pl.ds(off + j * 16, 16)]
                for k in range(16):
                    slot = k % RING
                    wait(tab, slot)
                    extract(vec[k], j * 16 + k, slot)
                    if k < 16 - RING:
                        fire(tab, vec[k + RING], slot)
                    else:
                        @pl.when(j < nrounds - 1)
                        def _(k=k, slot=slot, tab=tab, idx_v=idx_v,
                              j=j, off=off):
                            vecn = idx_v[pl.ds(off + (j + 1) * 16, 16)]
                            fire(tab, vecn[k - (16 - RING)], slot)
                        if nxt is not None:
                            @pl.when(j == nrounds - 1)
                            def _(k=k, slot=slot, nxt=nxt):
                                nidx, ntab, _, noff = nxt
                                vecn = nidx[pl.ds(noff, 16)]
                                fire(ntab, vecn[k - (16 - RING)], slot)
                return 0

            lax.fori_loop(0, nrounds, round_, 0)
            pltpu.sync_copy(rows_v, out.at[pl.ds(base + off, CHUNK)])

    return gather


def _mlp_body(u_ref, i_ref, w1_ref, b1_ref, w2_ref, b2_ref, w3_ref, b3_ref,
              o_ref):
    h = jnp.dot(u_ref[...], w1_ref[:EMBED, :],
                preferred_element_type=jnp.float32)
    h = h + jnp.dot(i_ref[...], w1_ref[EMBED:, :],
                    preferred_element_type=jnp.float32)
    h = jnp.maximum(h + b1_ref[...], 0.0)
    h = jnp.maximum(
        jnp.dot(h, w2_ref[...], preferred_element_type=jnp.float32)
        + b2_ref[...], 0.0)
    z = jnp.sum(h * w3_ref[...], axis=1) + b3_ref[0, 0]
    o_ref[...] = 1.0 / (1.0 + jnp.exp(-z))


def _mlp(u_emb, i_emb, W1, b1r, W2, b2r, w3r, b3r, block=2048):
    grid = BATCH // block
    return pl.pallas_call(
        _mlp_body,
        grid=(grid,),
        in_specs=[
            pl.BlockSpec((block, EMBED), lambda b: (b, 0)),
            pl.BlockSpec((block, EMBED), lambda b: (b, 0)),
            pl.BlockSpec((2 * EMBED, H1), lambda b: (0, 0)),
            pl.BlockSpec((1, H1), lambda b: (0, 0)),
            pl.BlockSpec((H1, H2), lambda b: (0, 0)),
            pl.BlockSpec((1, H2), lambda b: (0, 0)),
            pl.BlockSpec((1, H2), lambda b: (0, 0)),
            pl.BlockSpec((1, 1), lambda b: (0, 0)),
        ],
        out_specs=pl.BlockSpec((block,), lambda b: (b,)),
        out_shape=jax.ShapeDtypeStruct((BATCH,), jnp.float32),
    )(u_emb, i_emb, W1, b1r, W2, b2r, w3r, b3r)


def kernel(user_ids, item_ids, user_table, item_table, W1, b1, W2, b2, W3, b3):
    uid = user_ids.astype(jnp.int32)
    iid = item_ids.astype(jnp.int32)
    u_emb, i_emb = _build_gather()(uid, iid, user_table.T, item_table.T)
    return _mlp(u_emb, i_emb, W1, b1.reshape(1, H1), W2, b2.reshape(1, H2),
                W3.reshape(1, H2), b3.reshape(1, 1))
